# baseline (speedup 1.0000x reference)
import functools

import jax
import jax.numpy as jnp
from jax import lax
from jax.experimental import pallas as pl
from jax.experimental.pallas import tpu as pltpu
from jax.experimental.pallas import tpu_sc as plsc

_BLK = 8192
_INT_MAX = 2147483647
_NTILES = 16


def _tc_dist_block(data_ref, w_ref, bp_ref, d2_ref, *, n_rows, nb):
    i = pl.program_id(0)
    x = data_ref[...]
    e = jnp.dot(x, w_ref[...], preferred_element_type=jnp.float32)
    diff = e + bp_ref[...]
    d2 = jnp.sum(diff * diff, axis=1, keepdims=True)
    d2_ref[...] = d2

    @pl.when(i == nb - 1)
    def _():
        ridx = lax.broadcasted_iota(jnp.int32, (_BLK, 1), 0)
        valid = (i * _BLK + ridx) < n_rows
        d2_ref[...] = jnp.where(valid, d2, jnp.inf)


def _perm16(x, perm):
    dn = lax.GatherDimensionNumbers(
        offset_dims=(), collapsed_slice_dims=(0,), start_index_map=(0,))
    return lax.gather(x, perm[:, None], dn, (1,),
                      mode=lax.GatherScatterMode.PROMISE_IN_BOUNDS)


def _lex_min(bestv, besti, v, a):
    better = (v < bestv) | ((v == bestv) & (a < besti))
    return jnp.where(better, v, bestv), jnp.where(better, a, besti)


def _make_sc_argmin_gather(n_rows, n_pad, feat_dim):
    mesh = plsc.VectorSubcoreMesh(core_axis_name="c", subcore_axis_name="s")
    per_tile = n_pad // _NTILES
    n_chunks = per_tile // 16

    @functools.partial(
        pl.kernel,
        mesh=mesh,
        out_type=[
            jax.ShapeDtypeStruct((1, feat_dim), jnp.float32),
            jax.ShapeDtypeStruct((16,), jnp.int32),
        ],
        scratch_types=[
            pltpu.VMEM((per_tile,), jnp.float32),
            pltpu.VMEM((16,), jnp.float32),
            pltpu.VMEM((16,), jnp.int32),
            pltpu.VMEM((16 * _NTILES,), jnp.float32),
            pltpu.VMEM((16 * _NTILES,), jnp.int32),
            pltpu.VMEM_SHARED((16 * _NTILES,), jnp.float32),
            pltpu.VMEM_SHARED((16 * _NTILES,), jnp.int32),
            pltpu.VMEM((1, feat_dim), jnp.float32),
            pltpu.VMEM((16,), jnp.int32),
            pltpu.VMEM((16,), jnp.int32),
        ],
    )
    def sc_fn(d_hbm, data_hbm, lab_hbm, row_out, lab_out,
              d_v, tv_v, ti_v, allv_v, alli_v, shv_s, shi_s,
              row_v, lab_v, lsel_v):
        c = lax.axis_index("c")
        s = lax.axis_index("s")

        @pl.when(c == 0)
        def _():
            base = s * per_tile
            pltpu.sync_copy(d_hbm.at[pl.ds(base, per_tile)], d_v)
            lanes = lax.iota(jnp.int32, 16)

            def body(j, carry):
                bestv, besti = carry
                v = d_v[pl.ds(j * 16, 16)]
                a = lanes + (base + j * 16)
                return _lex_min(bestv, besti, v, a)

            bestv = d_v[pl.ds(0, 16)]
            besti = lanes + base
            bestv, besti = lax.fori_loop(1, n_chunks, body, (bestv, besti),
                                         unroll=4)
            for st in (8, 4, 2, 1):
                v = _perm16(bestv, lanes ^ st)
                a = _perm16(besti, lanes ^ st)
                bestv, besti = _lex_min(bestv, besti, v, a)
            tv_v[...] = bestv
            ti_v[...] = besti
            pltpu.sync_copy(tv_v, shv_s.at[pl.ds(s * 16, 16)])
            pltpu.sync_copy(ti_v, shi_s.at[pl.ds(s * 16, 16)])
            plsc.subcore_barrier()

            @pl.when(s == 0)
            def _():
                pltpu.sync_copy(shv_s, allv_v)
                pltpu.sync_copy(shi_s, alli_v)
                bv = allv_v[pl.ds(0, 16)]
                bi = alli_v[pl.ds(0, 16)]
                for t in range(1, _NTILES):
                    v = allv_v[pl.ds(t * 16, 16)]
                    a = alli_v[pl.ds(t * 16, 16)]
                    bv, bi = _lex_min(bv, bi, v, a)
                for st in (8, 4, 2, 1):
                    v = _perm16(bv, lanes ^ st)
                    a = _perm16(bi, lanes ^ st)
                    bv, bi = _lex_min(bv, bi, v, a)
                idx = bi[0]
                pltpu.sync_copy(data_hbm.at[pl.ds(idx, 1)], row_v)
                pltpu.sync_copy(row_v, row_out)
                wbase = jnp.minimum((idx // 8) * 8, jnp.int32(n_rows - 16))
                pltpu.sync_copy(lab_hbm.at[pl.ds(wbase, 16)], lab_v)
                sel = jnp.where(lanes == idx - wbase, lab_v[pl.ds(0, 16)], 0)
                for st in (8, 4, 2, 1):
                    sel = sel + _perm16(sel, lanes ^ st)
                lsel_v[...] = sel
                pltpu.sync_copy(lsel_v, lab_out)

    return sc_fn


def kernel(prototype_vector, data, label, W, b):
    n, feat = data.shape
    latent = W.shape[1]
    nb = (n + _BLK - 1) // _BLK
    n_pad = nb * _BLK

    bp = (b - prototype_vector).reshape(1, latent)

    d2 = pl.pallas_call(
        functools.partial(_tc_dist_block, n_rows=n, nb=nb),
        grid=(nb,),
        in_specs=[
            pl.BlockSpec((_BLK, feat), lambda i: (i, 0)),
            pl.BlockSpec((feat, latent), lambda i: (0, 0)),
            pl.BlockSpec((1, latent), lambda i: (0, 0)),
        ],
        out_specs=pl.BlockSpec((_BLK, 1), lambda i: (i, 0)),
        out_shape=jax.ShapeDtypeStruct((n_pad, 1), jnp.float32),
    )(data, W, bp)

    row, lab16 = _make_sc_argmin_gather(n, n_pad, feat)(
        d2.reshape(n_pad), data, label)
    return (row.reshape(feat), lab16[0])


# speedup vs baseline: 1.4815x; 1.4815x over previous
import functools

import jax
import jax.numpy as jnp
from jax import lax
from jax.experimental import pallas as pl
from jax.experimental.pallas import tpu as pltpu
from jax.experimental.pallas import tpu_sc as plsc

_BLK = 8192
_INT_MAX = 2147483647


def _tc_dist_block(data_ref, w_ref, bp_ref, idx_ref, bv_s, bi_s, *,
                   n_rows, nb):
    i = pl.program_id(0)
    x = data_ref[...]
    e = jnp.dot(x, w_ref[...], preferred_element_type=jnp.float32)
    diff = e + bp_ref[...]
    d2 = jnp.sum(diff * diff, axis=1, keepdims=True)
    ridx = lax.broadcasted_iota(jnp.int32, (_BLK, 1), 0)
    d2 = jnp.where((i * _BLK + ridx) < n_rows, d2, jnp.inf)
    m = jnp.min(d2)
    a = i * _BLK + jnp.min(jnp.where(d2 == m, ridx, _INT_MAX))

    @pl.when(i == 0)
    def _():
        bv_s[0] = jnp.float32(jnp.inf)
        bi_s[0] = jnp.int32(_INT_MAX)

    better = m < bv_s[0]
    bv_s[0] = jnp.where(better, m, bv_s[0])
    bi_s[0] = jnp.where(better, a, bi_s[0])

    @pl.when(i == nb - 1)
    def _():
        idx_ref[...] = jnp.full((1, 128), bi_s[0], jnp.int32)


def _make_sc_gather(n_rows, feat_dim):
    mesh = plsc.VectorSubcoreMesh(core_axis_name="c", subcore_axis_name="s")

    @functools.partial(
        pl.kernel,
        mesh=mesh,
        out_type=[
            jax.ShapeDtypeStruct((1, feat_dim), jnp.float32),
            jax.ShapeDtypeStruct((16,), jnp.int32),
        ],
        scratch_types=[
            pltpu.VMEM((128,), jnp.int32),
            pltpu.VMEM((1, feat_dim), jnp.float32),
            pltpu.VMEM((16,), jnp.int32),
            pltpu.VMEM((16,), jnp.int32),
        ],
    )
    def sc_fn(idx_hbm, data_hbm, lab_hbm, row_out, lab_out,
              idx_v, row_v, lab_v, lsel_v):
        c = lax.axis_index("c")
        s = lax.axis_index("s")

        @pl.when((c == 0) & (s == 0))
        def _():
            pltpu.sync_copy(idx_hbm, idx_v)
            idx = idx_v[pl.ds(0, 16)][0]
            pltpu.sync_copy(data_hbm.at[pl.ds(idx, 1)], row_v)
            pltpu.sync_copy(row_v, row_out)
            wbase = jnp.minimum((idx // 8) * 8, jnp.int32(n_rows - 16))
            pltpu.sync_copy(lab_hbm.at[pl.ds(wbase, 16)], lab_v)
            lanes = lax.iota(jnp.int32, 16)
            sel = jnp.where(lanes == idx - wbase, lab_v[pl.ds(0, 16)], 0)
            dn = lax.GatherDimensionNumbers(
                offset_dims=(), collapsed_slice_dims=(0,), start_index_map=(0,))
            for st in (8, 4, 2, 1):
                sel = sel + lax.gather(
                    sel, (lanes ^ st)[:, None], dn, (1,),
                    mode=lax.GatherScatterMode.PROMISE_IN_BOUNDS)
            lsel_v[...] = sel
            pltpu.sync_copy(lsel_v, lab_out)

    return sc_fn


def kernel(prototype_vector, data, label, W, b):
    n, feat = data.shape
    latent = W.shape[1]
    nb = (n + _BLK - 1) // _BLK

    bp = (b - prototype_vector).reshape(1, latent)

    idx128 = pl.pallas_call(
        functools.partial(_tc_dist_block, n_rows=n, nb=nb),
        grid=(nb,),
        in_specs=[
            pl.BlockSpec((_BLK, feat), lambda i: (i, 0)),
            pl.BlockSpec((feat, latent), lambda i: (0, 0)),
            pl.BlockSpec((1, latent), lambda i: (0, 0)),
        ],
        out_specs=pl.BlockSpec((1, 128), lambda i: (0, 0)),
        out_shape=jax.ShapeDtypeStruct((1, 128), jnp.int32),
        scratch_shapes=[
            pltpu.SMEM((1,), jnp.float32),
            pltpu.SMEM((1,), jnp.int32),
        ],
    )(data, W, bp)

    row, lab16 = _make_sc_gather(n, feat)(idx128.reshape(128), data, label)
    return (row.reshape(feat), lab16[0])


# speedup vs baseline: 1.6056x; 1.0838x over previous
import functools

import jax
import jax.numpy as jnp
from jax import lax
from jax.experimental import pallas as pl
from jax.experimental.pallas import tpu as pltpu
from jax.experimental.pallas import tpu_sc as plsc

_BLK = 8192
_INT_MAX = 2147483647


def _tc_dist_block(data_ref, w_ref, bp_ref, idx_ref, bv_s, bi_s, *,
                   n_rows, nb):
    i = pl.program_id(0)
    x = data_ref[...]
    e = jnp.dot(x, w_ref[...], preferred_element_type=jnp.float32)
    diff = e + bp_ref[...]
    d2 = jnp.sum(diff * diff, axis=1, keepdims=True)
    d2r = d2.reshape(_BLK // 128, 128)
    ridx = (lax.broadcasted_iota(jnp.int32, d2r.shape, 0) * 128
            + lax.broadcasted_iota(jnp.int32, d2r.shape, 1))
    d2r = jnp.where((i * _BLK + ridx) < n_rows, d2r, jnp.inf)
    m = jnp.min(d2r)
    a = i * _BLK + jnp.min(jnp.where(d2r == m, ridx, _INT_MAX))

    @pl.when(i == 0)
    def _():
        bv_s[0] = jnp.float32(jnp.inf)
        bi_s[0] = jnp.int32(_INT_MAX)

    better = m < bv_s[0]
    bv_s[0] = jnp.where(better, m, bv_s[0])
    bi_s[0] = jnp.where(better, a, bi_s[0])

    @pl.when(i == nb - 1)
    def _():
        idx_ref[...] = jnp.full((1, 128), bi_s[0], jnp.int32)


def _make_sc_gather(n_rows, feat_dim):
    mesh = plsc.VectorSubcoreMesh(core_axis_name="c", subcore_axis_name="s")

    @functools.partial(
        pl.kernel,
        mesh=mesh,
        out_type=[
            jax.ShapeDtypeStruct((1, feat_dim), jnp.float32),
            jax.ShapeDtypeStruct((16,), jnp.int32),
        ],
        scratch_types=[
            pltpu.VMEM((128,), jnp.int32),
            pltpu.VMEM((1, feat_dim), jnp.float32),
            pltpu.VMEM((16,), jnp.int32),
            pltpu.VMEM((16,), jnp.int32),
        ],
    )
    def sc_fn(idx_hbm, data_hbm, lab_hbm, row_out, lab_out,
              idx_v, row_v, lab_v, lsel_v):
        c = lax.axis_index("c")
        s = lax.axis_index("s")

        @pl.when((c == 0) & (s == 0))
        def _():
            pltpu.sync_copy(idx_hbm, idx_v)
            idx = idx_v[pl.ds(0, 16)][0]
            pltpu.sync_copy(data_hbm.at[pl.ds(idx, 1)], row_v)
            pltpu.sync_copy(row_v, row_out)
            wbase = jnp.minimum((idx // 8) * 8, jnp.int32(n_rows - 16))
            pltpu.sync_copy(lab_hbm.at[pl.ds(wbase, 16)], lab_v)
            lanes = lax.iota(jnp.int32, 16)
            sel = jnp.where(lanes == idx - wbase, lab_v[pl.ds(0, 16)], 0)
            dn = lax.GatherDimensionNumbers(
                offset_dims=(), collapsed_slice_dims=(0,), start_index_map=(0,))
            for st in (8, 4, 2, 1):
                sel = sel + lax.gather(
                    sel, (lanes ^ st)[:, None], dn, (1,),
                    mode=lax.GatherScatterMode.PROMISE_IN_BOUNDS)
            lsel_v[...] = sel
            pltpu.sync_copy(lsel_v, lab_out)

    return sc_fn


def kernel(prototype_vector, data, label, W, b):
    n, feat = data.shape
    latent = W.shape[1]
    nb = (n + _BLK - 1) // _BLK

    bp = (b - prototype_vector).reshape(1, latent)

    idx128 = pl.pallas_call(
        functools.partial(_tc_dist_block, n_rows=n, nb=nb),
        grid=(nb,),
        in_specs=[
            pl.BlockSpec((_BLK, feat), lambda i: (i, 0)),
            pl.BlockSpec((feat, latent), lambda i: (0, 0)),
            pl.BlockSpec((1, latent), lambda i: (0, 0)),
        ],
        out_specs=pl.BlockSpec((1, 128), lambda i: (0, 0)),
        out_shape=jax.ShapeDtypeStruct((1, 128), jnp.int32),
        scratch_shapes=[
            pltpu.SMEM((1,), jnp.float32),
            pltpu.SMEM((1,), jnp.int32),
        ],
    )(data, W, bp)

    row, lab16 = _make_sc_gather(n, feat)(idx128.reshape(128), data, label)
    return (row.reshape(feat), lab16[0])


# speedup vs baseline: 1.6478x; 1.0263x over previous
import functools

import jax
import jax.numpy as jnp
from jax import lax
from jax.experimental import pallas as pl
from jax.experimental.pallas import tpu as pltpu
from jax.experimental.pallas import tpu_sc as plsc

_BLK = 8192
_INT_MAX = 2147483647


def _tc_dist_block(data_ref, w_ref, b_ref, p_ref, idx_ref, bv_s, bi_s, *,
                   n_rows, nb):
    i = pl.program_id(0)
    x = data_ref[...]
    e = jnp.dot(x, w_ref[...], preferred_element_type=jnp.float32)
    diff = e + (b_ref[...] - p_ref[...])
    d2 = jnp.sum(diff * diff, axis=1, keepdims=True)
    d2r = d2.reshape(_BLK // 128, 128)
    ridx = (lax.broadcasted_iota(jnp.int32, d2r.shape, 0) * 128
            + lax.broadcasted_iota(jnp.int32, d2r.shape, 1))
    d2r = jnp.where((i * _BLK + ridx) < n_rows, d2r, jnp.inf)
    m = jnp.min(d2r)
    a = i * _BLK + jnp.min(jnp.where(d2r == m, ridx, _INT_MAX))

    @pl.when(i == 0)
    def _():
        bv_s[0] = jnp.float32(jnp.inf)
        bi_s[0] = jnp.int32(_INT_MAX)

    better = m < bv_s[0]
    bv_s[0] = jnp.where(better, m, bv_s[0])
    bi_s[0] = jnp.where(better, a, bi_s[0])

    @pl.when(i == nb - 1)
    def _():
        idx_ref[...] = jnp.full((1, 128), bi_s[0], jnp.int32)


def _make_sc_gather(n_rows, feat_dim):
    mesh = plsc.VectorSubcoreMesh(core_axis_name="c", subcore_axis_name="s")

    @functools.partial(
        pl.kernel,
        mesh=mesh,
        out_type=[
            jax.ShapeDtypeStruct((1, feat_dim), jnp.float32),
            jax.ShapeDtypeStruct((16,), jnp.int32),
        ],
        scratch_types=[
            pltpu.VMEM((128,), jnp.int32),
            pltpu.VMEM((1, feat_dim), jnp.float32),
            pltpu.VMEM((16,), jnp.int32),
            pltpu.VMEM((16,), jnp.int32),
        ],
    )
    def sc_fn(idx_hbm, data_hbm, lab_hbm, row_out, lab_out,
              idx_v, row_v, lab_v, lsel_v):
        c = lax.axis_index("c")
        s = lax.axis_index("s")

        @pl.when((c == 0) & (s == 0))
        def _():
            pltpu.sync_copy(idx_hbm, idx_v)
            idx = idx_v[pl.ds(0, 16)][0]
            pltpu.sync_copy(data_hbm.at[pl.ds(idx, 1)], row_v)
            pltpu.sync_copy(row_v, row_out)
            wbase = jnp.minimum((idx // 8) * 8, jnp.int32(n_rows - 16))
            pltpu.sync_copy(lab_hbm.at[pl.ds(wbase, 16)], lab_v)
            dn = lax.GatherDimensionNumbers(
                offset_dims=(), collapsed_slice_dims=(0,), start_index_map=(0,))
            lanepick = jnp.full((16,), idx - wbase, jnp.int32)
            lsel_v[...] = lax.gather(
                lab_v[pl.ds(0, 16)], lanepick[:, None], dn, (1,),
                mode=lax.GatherScatterMode.PROMISE_IN_BOUNDS)
            pltpu.sync_copy(lsel_v, lab_out)

    return sc_fn


def kernel(prototype_vector, data, label, W, b):
    n, feat = data.shape
    latent = W.shape[1]
    nb = (n + _BLK - 1) // _BLK

    idx128 = pl.pallas_call(
        functools.partial(_tc_dist_block, n_rows=n, nb=nb),
        grid=(nb,),
        in_specs=[
            pl.BlockSpec((_BLK, feat), lambda i: (i, 0)),
            pl.BlockSpec((feat, latent), lambda i: (0, 0)),
            pl.BlockSpec((1, latent), lambda i: (0, 0)),
            pl.BlockSpec((1, latent), lambda i: (0, 0)),
        ],
        out_specs=pl.BlockSpec((1, 128), lambda i: (0, 0)),
        out_shape=jax.ShapeDtypeStruct((1, 128), jnp.int32),
        scratch_shapes=[
            pltpu.SMEM((1,), jnp.float32),
            pltpu.SMEM((1,), jnp.int32),
        ],
    )(data, W, b.reshape(1, latent), prototype_vector.reshape(1, latent))

    row, lab16 = _make_sc_gather(n, feat)(idx128.reshape(128), data, label)
    return (row.reshape(feat), lab16[0])


# speedup vs baseline: 1.6782x; 1.0185x over previous
import functools

import jax
import jax.numpy as jnp
from jax import lax
from jax.experimental import pallas as pl
from jax.experimental.pallas import tpu as pltpu
from jax.experimental.pallas import tpu_sc as plsc

_BLK = 8192
_INT_MAX = 2147483647


def _tc_dist_block(data_ref, w_ref, b_ref, p_ref, idx_ref, bv_s, bi_s, *,
                   n_rows, nb):
    i = pl.program_id(0)
    x = data_ref[...]
    e = jnp.dot(x, w_ref[...], preferred_element_type=jnp.float32)
    diff = e + (b_ref[...] - p_ref[...])
    d2 = jnp.sum(diff * diff, axis=1, keepdims=True)
    d2r = d2.reshape(_BLK // 128, 128)
    ridx = (lax.broadcasted_iota(jnp.int32, d2r.shape, 0) * 128
            + lax.broadcasted_iota(jnp.int32, d2r.shape, 1))
    d2r = jnp.where((i * _BLK + ridx) < n_rows, d2r, jnp.inf)
    m = jnp.min(d2r)
    a = i * _BLK + jnp.min(jnp.where(d2r == m, ridx, _INT_MAX))

    @pl.when(i == 0)
    def _():
        bv_s[0] = jnp.float32(jnp.inf)
        bi_s[0] = jnp.int32(_INT_MAX)

    better = m < bv_s[0]
    bv_s[0] = jnp.where(better, m, bv_s[0])
    bi_s[0] = jnp.where(better, a, bi_s[0])

    @pl.when(i == nb - 1)
    def _():
        idx_ref[...] = jnp.full((1, 128), bi_s[0], jnp.int32)


def _make_sc_gather(n_rows, feat_dim):
    mesh = plsc.VectorSubcoreMesh(core_axis_name="c", subcore_axis_name="s",
                                  num_cores=1)

    @functools.partial(
        pl.kernel,
        mesh=mesh,
        out_type=[
            jax.ShapeDtypeStruct((1, feat_dim), jnp.float32),
            jax.ShapeDtypeStruct((16,), jnp.int32),
        ],
        scratch_types=[
            pltpu.VMEM((128,), jnp.int32),
            pltpu.VMEM((1, feat_dim), jnp.float32),
            pltpu.VMEM((16,), jnp.int32),
            pltpu.VMEM((16,), jnp.int32),
        ],
    )
    def sc_fn(idx_hbm, data_hbm, lab_hbm, row_out, lab_out,
              idx_v, row_v, lab_v, lsel_v):
        c = lax.axis_index("c")
        s = lax.axis_index("s")

        @pl.when((c == 0) & (s == 0))
        def _():
            pltpu.sync_copy(idx_hbm, idx_v)
            idx = idx_v[pl.ds(0, 16)][0]
            pltpu.sync_copy(data_hbm.at[pl.ds(idx, 1)], row_v)
            pltpu.sync_copy(row_v, row_out)
            wbase = jnp.minimum((idx // 8) * 8, jnp.int32(n_rows - 16))
            pltpu.sync_copy(lab_hbm.at[pl.ds(wbase, 16)], lab_v)
            dn = lax.GatherDimensionNumbers(
                offset_dims=(), collapsed_slice_dims=(0,), start_index_map=(0,))
            lanepick = jnp.full((16,), idx - wbase, jnp.int32)
            lsel_v[...] = lax.gather(
                lab_v[pl.ds(0, 16)], lanepick[:, None], dn, (1,),
                mode=lax.GatherScatterMode.PROMISE_IN_BOUNDS)
            pltpu.sync_copy(lsel_v, lab_out)

    return sc_fn


def kernel(prototype_vector, data, label, W, b):
    n, feat = data.shape
    latent = W.shape[1]
    nb = (n + _BLK - 1) // _BLK

    idx128 = pl.pallas_call(
        functools.partial(_tc_dist_block, n_rows=n, nb=nb),
        grid=(nb,),
        in_specs=[
            pl.BlockSpec((_BLK, feat), lambda i: (i, 0)),
            pl.BlockSpec((feat, latent), lambda i: (0, 0)),
            pl.BlockSpec((1, latent), lambda i: (0, 0)),
            pl.BlockSpec((1, latent), lambda i: (0, 0)),
        ],
        out_specs=pl.BlockSpec((1, 128), lambda i: (0, 0)),
        out_shape=jax.ShapeDtypeStruct((1, 128), jnp.int32),
        scratch_shapes=[
            pltpu.SMEM((1,), jnp.float32),
            pltpu.SMEM((1,), jnp.int32),
        ],
    )(data, W, b.reshape(1, latent), prototype_vector.reshape(1, latent))

    row, lab16 = _make_sc_gather(n, feat)(idx128.reshape(128), data, label)
    return (row.reshape(feat), lab16[0])


# speedup vs baseline: 2.2616x; 1.3476x over previous
import functools

import jax
import jax.numpy as jnp
from jax import lax
from jax.experimental import pallas as pl
from jax.experimental.pallas import tpu as pltpu

_BLK = 8192
_INT_MAX = 2147483647


def _tc_kernel(data_ref, w_ref, b_ref, p_ref, data_any, lab_any,
               row_ref, lab_ref, bv_s, bi_s, row_v, lab_s, sem, sem2, *,
               n_rows, nb):
    i = pl.program_id(0)
    x = data_ref[...]
    e = jnp.dot(x, w_ref[...], preferred_element_type=jnp.float32)
    diff = e + (b_ref[...] - p_ref[...])
    d2 = jnp.sum(diff * diff, axis=1, keepdims=True)
    d2r = d2.reshape(_BLK // 128, 128)
    ridx = (lax.broadcasted_iota(jnp.int32, d2r.shape, 0) * 128
            + lax.broadcasted_iota(jnp.int32, d2r.shape, 1))
    d2r = jnp.where((i * _BLK + ridx) < n_rows, d2r, jnp.inf)
    m = jnp.min(d2r)
    a = i * _BLK + jnp.min(jnp.where(d2r == m, ridx, _INT_MAX))

    @pl.when(i == 0)
    def _():
        bv_s[0] = jnp.float32(jnp.inf)
        bi_s[0] = jnp.int32(_INT_MAX)

    better = m < bv_s[0]
    bv_s[0] = jnp.where(better, m, bv_s[0])
    bi_s[0] = jnp.where(better, a, bi_s[0])

    @pl.when(i == nb - 1)
    def _():
        idx = bi_s[0]
        rbase = (idx // 8) * 8
        wbase = (idx // 128) * 128
        pltpu.make_async_copy(
            data_any.at[pl.ds(rbase, 8)], row_v, sem).start()
        pltpu.make_async_copy(
            lab_any.at[pl.ds(wbase, 128)], lab_s, sem2).start()
        pltpu.make_async_copy(
            data_any.at[pl.ds(rbase, 8)], row_v, sem).wait()
        pltpu.make_async_copy(
            lab_any.at[pl.ds(wbase, 128)], lab_s, sem2).wait()
        rows = row_v[...]
        rmask = lax.broadcasted_iota(jnp.int32, rows.shape, 0) == idx - rbase
        row_ref[...] = jnp.sum(jnp.where(rmask, rows, 0.0), axis=0,
                               keepdims=True)
        lab_ref[0, 0] = lab_s[idx - wbase]


def kernel(prototype_vector, data, label, W, b):
    n, feat = data.shape
    latent = W.shape[1]
    nb = (n + _BLK - 1) // _BLK

    row, lab = pl.pallas_call(
        functools.partial(_tc_kernel, n_rows=n, nb=nb),
        grid=(nb,),
        in_specs=[
            pl.BlockSpec((_BLK, feat), lambda i: (i, 0)),
            pl.BlockSpec((feat, latent), lambda i: (0, 0)),
            pl.BlockSpec((1, latent), lambda i: (0, 0)),
            pl.BlockSpec((1, latent), lambda i: (0, 0)),
            pl.BlockSpec(memory_space=pltpu.MemorySpace.HBM),
            pl.BlockSpec(memory_space=pltpu.MemorySpace.HBM),
        ],
        out_specs=[
            pl.BlockSpec((1, feat), lambda i: (0, 0)),
            pl.BlockSpec((1, 1), lambda i: (0, 0), memory_space=pltpu.SMEM),
        ],
        out_shape=[
            jax.ShapeDtypeStruct((1, feat), jnp.float32),
            jax.ShapeDtypeStruct((1, 1), jnp.int32),
        ],
        scratch_shapes=[
            pltpu.SMEM((1,), jnp.float32),
            pltpu.SMEM((1,), jnp.int32),
            pltpu.VMEM((8, feat), jnp.float32),
            pltpu.SMEM((128,), jnp.int32),
            pltpu.SemaphoreType.DMA,
            pltpu.SemaphoreType.DMA,
        ],
    )(data, W, b.reshape(1, latent), prototype_vector.reshape(1, latent),
      data, jnp.pad(label, (0, 128)))
    return (row.reshape(feat), lab[0, 0])
